# Initial kernel scaffold; baseline (speedup 1.0000x reference)
import jax
import jax.numpy as jnp
from jax.experimental import pallas as pl


def kernel(inputs, input_timestamp, input_id, epoch, static_graph, emb, Wg, bg, W1, b1, W2, b2):
    raise NotImplementedError("write your pallas kernel here")


# speedup vs baseline: 4.6344x; 4.6344x over previous
import functools

import jax
import jax.numpy as jnp
from jax import lax
from jax.experimental import pallas as pl
from jax.experimental.pallas import tpu as pltpu
from jax.experimental.pallas import tpu_sc as plsc

NC = 2
NS = 16
NW = NC * NS
CHUNK = 128


def _cdiv(a, b):
    return (a + b - 1) // b


def _gate_body(emb_ref, wg_ref, bg_ref, out_ref):
    x = emb_ref[...]
    out_ref[...] = x * jax.nn.sigmoid(x @ wg_ref[...] + bg_ref[...])


def _gate(emb, Wg, bg2, bm):
    N, D = emb.shape
    return pl.pallas_call(
        _gate_body,
        grid=(N // bm,),
        in_specs=[
            pl.BlockSpec((bm, D), lambda i: (i, 0)),
            pl.BlockSpec((D, D), lambda i: (0, 0)),
            pl.BlockSpec((1, D), lambda i: (0, 0)),
        ],
        out_specs=pl.BlockSpec((bm, D), lambda i: (i, 0)),
        out_shape=jax.ShapeDtypeStruct((N, D), jnp.float32),
    )(emb, Wg, bg2)


def _combine_body(relu, p_ref, d_ref, w_ref, b_ref, out_ref):
    p = p_ref[...]
    d = d_ref[...]
    s = p[0] + p[1]
    deg = d[0] + d[1]
    inv = 1.0 / jnp.maximum(deg, 1.0)
    y = (s * inv) @ w_ref[...] + b_ref[...]
    if relu:
        y = jnp.maximum(y, 0.0)
    out_ref[...] = y


def _combine(partials, degs, W, b2, relu, N, bm):
    D = W.shape[0]
    body = functools.partial(_combine_body, relu)
    return pl.pallas_call(
        body,
        grid=(N // bm,),
        in_specs=[
            pl.BlockSpec((NC, bm, D), lambda i: (0, i, 0)),
            pl.BlockSpec((NC, bm, D), lambda i: (0, i, 0)),
            pl.BlockSpec((D, D), lambda i: (0, 0)),
            pl.BlockSpec((1, D), lambda i: (0, 0)),
        ],
        out_specs=pl.BlockSpec((bm, D), lambda i: (i, 0)),
        out_shape=jax.ShapeDtypeStruct((N, D), jnp.float32),
    )(partials, degs, W, b2)


def _make_segsum(N, D, cpw, Np, rpw):
    mesh = plsc.VectorSubcoreMesh(core_axis_name="c", subcore_axis_name="s")

    @functools.partial(
        pl.kernel,
        out_type=jax.ShapeDtypeStruct((NC, Np, D), jnp.float32),
        mesh=mesh,
        scratch_types=[
            pltpu.VMEM((cpw, CHUNK), jnp.int32),
            pltpu.VMEM((cpw, CHUNK), jnp.int32),
            pltpu.VMEM((CHUNK, D), jnp.float32),
            pltpu.VMEM_SHARED((Np, D), jnp.float32),
        ],
    )
    def seg(table_hbm, src_hbm, dst_hbm, z_hbm, out_hbm,
            srcbuf, dstbuf, rowbuf, acc):
        core = lax.axis_index("c")
        sub = lax.axis_index("s")
        wid = sub * NC + core
        pltpu.sync_copy(z_hbm, acc.at[pl.ds(sub * rpw, rpw)])
        pltpu.sync_copy(src_hbm.at[wid], srcbuf)
        pltpu.sync_copy(dst_hbm.at[wid], dstbuf)
        plsc.subcore_barrier()

        @pl.loop(0, cpw)
        def _(j):
            pltpu.sync_copy(table_hbm.at[srcbuf.at[j]], rowbuf)
            pltpu.sync_copy(rowbuf, acc.at[dstbuf.at[j]], add=True)

        plsc.subcore_barrier()
        pltpu.sync_copy(acc.at[pl.ds(sub * rpw, rpw)],
                        out_hbm.at[core, pl.ds(sub * rpw, rpw)])

    return seg


def _make_degree(cpw, Np, rpw):
    mesh = plsc.VectorSubcoreMesh(core_axis_name="c", subcore_axis_name="s")

    @functools.partial(
        pl.kernel,
        out_type=jax.ShapeDtypeStruct((NC, Np, CHUNK), jnp.float32),
        mesh=mesh,
        scratch_types=[
            pltpu.VMEM((cpw, CHUNK), jnp.int32),
            pltpu.VMEM((CHUNK, CHUNK), jnp.float32),
            pltpu.VMEM_SHARED((Np, CHUNK), jnp.float32),
        ],
    )
    def degk(dst_hbm, ones_hbm, z_hbm, out_hbm, dstbuf, onesbuf, acc):
        core = lax.axis_index("c")
        sub = lax.axis_index("s")
        wid = sub * NC + core
        pltpu.sync_copy(z_hbm, acc.at[pl.ds(sub * rpw, rpw)])
        pltpu.sync_copy(dst_hbm.at[wid], dstbuf)
        pltpu.sync_copy(ones_hbm, onesbuf)
        plsc.subcore_barrier()

        @pl.loop(0, cpw)
        def _(j):
            pltpu.sync_copy(onesbuf, acc.at[dstbuf.at[j]], add=True)

        plsc.subcore_barrier()
        pltpu.sync_copy(acc.at[pl.ds(sub * rpw, rpw)],
                        out_hbm.at[core, pl.ds(sub * rpw, rpw)])

    return degk


def _make_lookup(D, lpw, BLp):
    mesh = plsc.VectorSubcoreMesh(core_axis_name="c", subcore_axis_name="s")

    @functools.partial(
        pl.kernel,
        out_type=jax.ShapeDtypeStruct((BLp, D), jnp.float32),
        mesh=mesh,
        scratch_types=[
            pltpu.VMEM((lpw, CHUNK), jnp.int32),
            pltpu.VMEM((CHUNK, D), jnp.float32),
        ],
    )
    def lk(table_hbm, idx_hbm, out_hbm, idxbuf, rowbuf):
        core = lax.axis_index("c")
        sub = lax.axis_index("s")
        wid = sub * NC + core
        pltpu.sync_copy(idx_hbm.at[wid], idxbuf)

        @pl.loop(0, lpw)
        def _(j):
            pltpu.sync_copy(table_hbm.at[idxbuf.at[j]], rowbuf)
            pltpu.sync_copy(rowbuf, out_hbm.at[pl.ds((wid * lpw + j) * CHUNK,
                                                     CHUNK)])

    return lk


def kernel(inputs, input_timestamp, input_id, epoch, static_graph,
           emb, Wg, bg, W1, b1, W2, b2):
    N, D = emb.shape
    E = static_graph.shape[1]
    B, L = inputs.shape
    bm = 1000 if N % 1000 == 0 else 8

    cpw = _cdiv(E, NW * CHUNK)
    E_pad = cpw * NW * CHUNK
    src = static_graph[0].astype(jnp.int32)
    dst = static_graph[1].astype(jnp.int32)
    src_p = jnp.concatenate(
        [src, jnp.zeros((E_pad - E,), jnp.int32)]).reshape(NW, cpw, CHUNK)
    dst_p = jnp.concatenate(
        [dst, jnp.full((E_pad - E,), N, jnp.int32)]).reshape(NW, cpw, CHUNK)

    Np = _cdiv(N + 1, 8 * NS) * 8 * NS
    rpw = Np // NS
    zer = jnp.zeros((rpw, D), jnp.float32)
    ones = jnp.ones((CHUNK, CHUNK), jnp.float32)

    bg2 = bg.reshape(1, D)
    b12 = b1.reshape(1, D)
    b22 = b2.reshape(1, D)

    segsum = _make_segsum(N, D, cpw, Np, rpw)

    degs = _make_degree(cpw, Np, rpw)(dst_p, ones, zer)
    h = _gate(emb, Wg, bg2, bm)
    part1 = segsum(h, src_p, dst_p, zer)
    h1 = _combine(part1, degs, W1, b12, True, N, bm)
    part2 = segsum(h1, src_p, dst_p, zer)
    all_st = _combine(part2, degs, W2, b22, False, N, bm)

    BL = B * L
    lpw = _cdiv(BL, NW * CHUNK)
    BLp = lpw * NW * CHUNK
    idx = inputs.reshape(-1).astype(jnp.int32)
    idx = jnp.concatenate(
        [idx, jnp.zeros((BLp - BL,), jnp.int32)]).reshape(NW, lpw, CHUNK)
    rows = _make_lookup(D, lpw, BLp)(all_st, idx)
    user_st_seq_rep = rows[:BL].reshape(B, L, D)
    return (user_st_seq_rep, all_st)
